# baseline (speedup 1.0000x reference)
import functools

import jax
import jax.numpy as jnp
import numpy as np
from jax import lax
from jax.experimental import pallas as pl
from jax.experimental.pallas import tpu as pltpu
from jax.experimental.pallas import tpu_sc as plsc

_DIM = 32
_SCALE = float(np.sqrt(_DIM))

_NC, _NS = 2, 16
_NW = _NC * _NS

_L = 128
_S = 8
_A = _DIM // _S
_G = 4
_U_ROWS = _G * _L


def _sc_gather_scale(xq, table, n_rows, n_cols):
    n_i0t = n_rows // _L
    gblocks = n_i0t // _G
    n_units = n_cols * gblocks
    units_per_w = n_units // _NW
    assert n_units % _NW == 0

    mesh = plsc.VectorSubcoreMesh(
        core_axis_name="c", subcore_axis_name="s",
        num_cores=_NC, num_subcores=_NS,
    )

    @functools.partial(
        pl.kernel,
        out_type=jax.ShapeDtypeStruct((n_cols, _A, n_i0t, _S, _L),
                                      jnp.float32),
        mesh=mesh,
        scratch_types=[
            pltpu.VMEM((_G, _L), jnp.int32),
            pltpu.VMEM((_U_ROWS, _DIM), jnp.float32),
            pltpu.VMEM((_A, _G, _S, _L), jnp.float32),
            pltpu.SemaphoreType.DMA,
        ],
        compiler_params=pltpu.CompilerParams(
            use_tc_tiling_on_sc=False, needs_layout_passes=False),
    )
    def k(xq_hbm, table_hbm, outq_hbm, idx_v, rows_v, stage_v, gsem):
        wid = lax.axis_index("s") * _NC + lax.axis_index("c")
        u_base = wid * units_per_w
        iota = lax.iota(jnp.int32, 16)

        def unit_body(uu, carry):
            u = u_base + uu
            i1 = u // gblocks
            g0 = (u % gblocks) * _G
            rt, s1 = i1 // _S, i1 % _S

            for j in range(_G):
                pltpu.sync_copy(xq_hbm.at[rt, g0 + j, s1], idx_v.at[j])
            copies = []
            for j in range(_G):
                copies.append(pltpu.async_copy(
                    table_hbm.at[idx_v.at[j]],
                    rows_v.at[pl.ds(j * _L, _L)],
                    gsem,
                ))
            for cp in copies:
                cp.wait()

            def tr_body(q, carry2):
                gg = q // 8
                lv = q % 8
                row_vec = jnp.full((16,), 16, jnp.int32) * q + iota
                for c in range(_DIM):
                    col_vec = jnp.full((16,), c, jnp.int32)
                    vals = plsc.load_gather(rows_v, [row_vec, col_vec])
                    stage_v[c // _S, gg, c % _S, pl.ds(lv * 16, 16)] = (
                        vals * _SCALE)
                return carry2

            lax.fori_loop(0, _G * _S, tr_body, 0)

            for a in range(_A):
                pltpu.sync_copy(stage_v.at[a],
                                outq_hbm.at[i1, a, pl.ds(g0, _G)])
            return carry

        lax.fori_loop(0, units_per_w, unit_body, 0)

    return k(xq, table)


def kernel(x, table):
    n_rows, n_cols = x.shape
    xq = x.reshape(n_rows // _L, _L, n_cols // _S, _S).transpose(2, 0, 3, 1)
    outq = _sc_gather_scale(xq, table, n_rows, n_cols)
    out = outq.transpose(2, 4, 0, 1, 3).reshape(n_rows, n_cols, _DIM)
    return out


# speedup vs baseline: 1.6903x; 1.6903x over previous
import functools

import jax
import jax.numpy as jnp
import numpy as np
from jax import lax
from jax.experimental import pallas as pl
from jax.experimental.pallas import tpu as pltpu
from jax.experimental.pallas import tpu_sc as plsc

_DIM = 32
_SCALE = float(np.sqrt(_DIM))

_NC, _NS = 2, 16
_NW = _NC * _NS

_L = 128
_S = 8
_A = _DIM // _S
_G = 8
_U_ROWS = _G * _L


def _sc_gather_scale(xq, table, n_rows, n_cols):
    n_i0t = n_rows // _L
    gblocks = n_i0t // _G
    n_units = n_cols * gblocks
    units_per_w = n_units // _NW
    assert n_units % _NW == 0

    mesh = plsc.VectorSubcoreMesh(
        core_axis_name="c", subcore_axis_name="s",
        num_cores=_NC, num_subcores=_NS,
    )

    @functools.partial(
        pl.kernel,
        out_type=jax.ShapeDtypeStruct((n_cols, _A, n_i0t * _S * _L),
                                      jnp.float32),
        mesh=mesh,
        scratch_types=[
            pltpu.VMEM((_G, _S, _L), jnp.int32),
            pltpu.VMEM((_U_ROWS, _DIM), jnp.float32),
            pltpu.VMEM((_A * _G * _S * _L,), jnp.float32),
            pltpu.SemaphoreType.DMA,
        ],
        compiler_params=pltpu.CompilerParams(
            use_tc_tiling_on_sc=False, needs_layout_passes=False),
    )
    def k(xq_hbm, table_hbm, outq_hbm, idx_v, rows_v, stage_v, gsem):
        wid = lax.axis_index("s") * _NC + lax.axis_index("c")
        u_base = wid * units_per_w
        iota = lax.iota(jnp.int32, 16)
        half_pat = [
            jnp.full((16,), h * 16, jnp.int32) + iota for h in (0, 1)]
        half_const = [
            (hp // _S) * (_G * _S * _L) + (hp % _S) * _L for hp in half_pat]
        stage_flat_shape = _A * _G * _S * _L

        def unit_body(uu, carry):
            u = u_base + uu
            i1 = u // gblocks
            g0 = (u % gblocks) * _G
            rt, s1 = i1 // _S, i1 % _S

            pltpu.sync_copy(xq_hbm.at[rt, pl.ds(g0, _G)], idx_v)
            copies = []
            for j in range(_G):
                copies.append(pltpu.async_copy(
                    table_hbm.at[idx_v.at[j, s1]],
                    rows_v.at[pl.ds(j * _L, _L)],
                    gsem,
                ))
            for cp in copies:
                cp.wait()

            @plsc.parallel_loop(0, _U_ROWS, 1, unroll=4)
            def tr_body(p):
                gg = p // _L
                l = p % _L
                base = jnp.full((16,), gg * (_S * _L) + l, jnp.int32)
                for h in (0, 1):
                    vals = rows_v[p, pl.ds(h * 16, 16)] * _SCALE
                    plsc.store_scatter(stage_v, [base + half_const[h]], vals)

            run = _G * _S * _L
            for a in range(_A):
                pltpu.sync_copy(stage_v.at[pl.ds(a * run, run)],
                                outq_hbm.at[i1, a, pl.ds(g0 * _S * _L, run)])
            return carry

        lax.fori_loop(0, units_per_w, unit_body, 0)

    return k(xq, table)


def kernel(x, table):
    n_rows, n_cols = x.shape
    xq = x.reshape(n_rows // _L, _L, n_cols // _S, _S).transpose(2, 0, 3, 1)
    outq = _sc_gather_scale(xq, table, n_rows, n_cols)
    outq = outq.reshape(n_cols, _A, n_rows // _L, _S, _L)
    out = outq.transpose(2, 4, 0, 1, 3).reshape(n_rows, n_cols, _DIM)
    return out


# speedup vs baseline: 2.0868x; 1.2346x over previous
import functools

import jax
import jax.numpy as jnp
import numpy as np
from jax import lax
from jax.experimental import pallas as pl
from jax.experimental.pallas import tpu as pltpu
from jax.experimental.pallas import tpu_sc as plsc

_DIM = 32
_SCALE = float(np.sqrt(_DIM))

_NC, _NS = 2, 16
_NW = _NC * _NS

_L = 128
_S = 8
_A = _DIM // _S
_G = 4
_U_ROWS = _G * _L
_RUN = _G * _S * _L
_STAGE = _A * _RUN


def _sc_gather_scale(xq, table, n_rows, n_cols):
    n_i0t = n_rows // _L
    gblocks = n_i0t // _G
    n_units = n_cols * gblocks
    units_per_w = n_units // _NW
    assert n_units % _NW == 0 and units_per_w % 2 == 0

    mesh = plsc.VectorSubcoreMesh(
        core_axis_name="c", subcore_axis_name="s",
        num_cores=_NC, num_subcores=_NS,
    )

    @functools.partial(
        pl.kernel,
        out_type=jax.ShapeDtypeStruct((n_cols, _A, n_i0t * _S * _L),
                                      jnp.float32),
        mesh=mesh,
        scratch_types=[
            pltpu.VMEM((_G, _S, _L), jnp.int32),
            pltpu.VMEM((_G, _S, _L), jnp.int32),
            pltpu.VMEM((_U_ROWS, _DIM), jnp.float32),
            pltpu.VMEM((_U_ROWS, _DIM), jnp.float32),
            pltpu.VMEM((_STAGE,), jnp.float32),
            pltpu.VMEM((_STAGE,), jnp.float32),
            pltpu.SemaphoreType.DMA,
            pltpu.SemaphoreType.DMA,
            pltpu.SemaphoreType.DMA,
            pltpu.SemaphoreType.DMA,
            pltpu.SemaphoreType.DMA,
            pltpu.SemaphoreType.DMA,
        ],
        compiler_params=pltpu.CompilerParams(
            use_tc_tiling_on_sc=False, needs_layout_passes=False),
    )
    def k(xq_hbm, table_hbm, outq_hbm, idx0, idx1, rows0, rows1,
          stage0, stage1, isem0, isem1, gsem0, gsem1, osem0, osem1):
        wid = lax.axis_index("s") * _NC + lax.axis_index("c")
        u_base = wid * units_per_w
        idx_bufs = (idx0, idx1)
        row_bufs = (rows0, rows1)
        stage_bufs = (stage0, stage1)
        isems = (isem0, isem1)
        gsems = (gsem0, gsem1)
        osems = (osem0, osem1)
        iota = lax.iota(jnp.int32, 16)
        half_pat = [
            jnp.full((16,), h * 16, jnp.int32) + iota for h in (0, 1)]
        half_const = [
            (hp // _S) * _RUN + (hp % _S) * _L for hp in half_pat]

        def unit_pos(u):
            i1 = u // gblocks
            g0 = (u % gblocks) * _G
            return i1, g0

        def fire_idx(u, buf):
            i1, g0 = unit_pos(u_base + u)
            pltpu.async_copy(
                xq_hbm.at[i1 // _S, pl.ds(g0, _G)], idx_bufs[buf],
                isems[buf])

        def wait_idx(buf):
            pltpu.make_async_copy(
                xq_hbm.at[0, pl.ds(0, _G)], idx_bufs[buf], isems[buf],
            ).wait()

        def fire_gathers(u, buf):
            i1, _ = unit_pos(u_base + u)
            s1 = i1 % _S
            for j in range(_G):
                pltpu.async_copy(
                    table_hbm.at[idx_bufs[buf].at[j, s1]],
                    row_bufs[buf].at[pl.ds(j * _L, _L)],
                    gsems[buf],
                )

        def drain_gathers(buf):
            pltpu.make_async_copy(
                table_hbm.at[pl.ds(0, _U_ROWS)], row_bufs[buf], gsems[buf],
            ).wait()

        def fire_out(u, buf):
            i1, g0 = unit_pos(u_base + u)
            woff = g0 * _S * _L
            for a in range(_A):
                pltpu.async_copy(
                    stage_bufs[buf].at[pl.ds(a * _RUN, _RUN)],
                    outq_hbm.at[i1, a, pl.ds(woff, _RUN)],
                    osems[buf])

        def drain_out(buf):
            pltpu.make_async_copy(
                stage_bufs[buf], outq_hbm.at[0, 0, pl.ds(0, _STAGE)],
                osems[buf],
            ).wait()

        def transpose(buf):
            rows_v = row_bufs[buf]
            stage_v = stage_bufs[buf]

            @plsc.parallel_loop(0, _U_ROWS, 1, unroll=4)
            def tr_body(p):
                gg = p // _L
                l = p % _L
                base = jnp.full((16,), gg * (_S * _L) + l, jnp.int32)
                for h in (0, 1):
                    vals = rows_v[p, pl.ds(h * 16, 16)] * _SCALE
                    plsc.store_scatter(stage_v, [base + half_const[h]], vals)

        pltpu.sync_copy(xq_hbm.at[unit_pos(u_base)[0] // _S,
                                  pl.ds(unit_pos(u_base)[1], _G)], idx0)
        fire_gathers(0, 0)
        fire_idx(1, 1)

        n = units_per_w

        def pair_body(g, carry):
            for b in (0, 1):
                u = g * 2 + b
                nb = 1 - b
                @pl.when(u + 1 < n)
                def _():
                    wait_idx(nb)
                @pl.when(u >= 2)
                def _():
                    drain_out(b)
                @pl.when(u + 1 < n)
                def _():
                    fire_gathers(u + 1, nb)
                drain_gathers(b)
                @pl.when(u + 2 < n)
                def _():
                    fire_idx(u + 2, b)
                transpose(b)
                fire_out(u, b)
            return carry

        lax.fori_loop(0, n // 2, pair_body, 0)
        drain_out(0)
        drain_out(1)

    return k(xq, table)


def kernel(x, table):
    n_rows, n_cols = x.shape
    xq = x.reshape(n_rows // _L, _L, n_cols // _S, _S).transpose(2, 0, 3, 1)
    outq = _sc_gather_scale(xq, table, n_rows, n_cols)
    outq = outq.reshape(n_cols, _A, n_rows // _L, _S, _L)
    out = outq.transpose(2, 4, 0, 1, 3).reshape(n_rows, n_cols, _DIM)
    return out
